# baseline (speedup 1.0000x reference)
import functools

import jax
import jax.numpy as jnp
from jax import lax
from jax.experimental import pallas as pl
from jax.experimental.pallas import tpu as pltpu
from jax.experimental.pallas import tpu_sc as plsc

F = 128
NW = 32
CHUNK = 80


def _silu(x):
    return x * jax.nn.sigmoid(x)


def _atoms_body(atom_ref, w1_ref, w2_ref, w3_ref, p1_ref, p2_ref, p3_ref):
    a = atom_ref[...]
    p1_ref[...] = jnp.dot(a, w1_ref[...], preferred_element_type=jnp.float32)
    p2_ref[...] = jnp.dot(a, w2_ref[...], preferred_element_type=jnp.float32)
    p3_ref[...] = jnp.dot(a, w3_ref[...], preferred_element_type=jnp.float32)


def _edges_body(edge_ref, w4_ref, w5_ref, q4_ref, q5_ref):
    e = edge_ref[...]
    q4_ref[...] = jnp.dot(e, w4_ref[...], preferred_element_type=jnp.float32)
    q5_ref[...] = jnp.dot(e, w5_ref[...], preferred_element_type=jnp.float32)


def _sc_index_prep(nbr0, nbr1, bp0, bp1):
    L = bp0.shape[0]
    rows_per_w = L // NW
    n_blk = rows_per_w // CHUNK
    slot = 5 * CHUNK
    words_per_w = n_blk * slot
    mesh = plsc.VectorSubcoreMesh(core_axis_name="c", subcore_axis_name="s")

    @functools.partial(
        pl.kernel,
        mesh=mesh,
        out_type=jax.ShapeDtypeStruct((NW * words_per_w,), jnp.int32),
        scratch_types=[
            pltpu.VMEM((words_per_w,), jnp.int32),
            pltpu.SemaphoreType.DMA,
            pltpu.SemaphoreType.DMA,
        ],
    )
    def idx_kernel(nbr0_hbm, nbr1_hbm, bp0_hbm, bp1_hbm, idx5_hbm,
                   ibuf, sem_a, sem_g):
        wid = lax.axis_index("s") * 2 + lax.axis_index("c")
        base = wid * rows_per_w

        def cp(b, carry):
            src = pl.ds(base + b * CHUNK, CHUNK)
            pltpu.make_async_copy(bp0_hbm.at[src],
                                  ibuf.at[pl.ds(b * slot + 3 * CHUNK,
                                                CHUNK)], sem_a).start()
            pltpu.make_async_copy(bp1_hbm.at[src],
                                  ibuf.at[pl.ds(b * slot + 4 * CHUNK,
                                                CHUNK)], sem_a).start()
            return carry

        lax.fori_loop(0, n_blk, cp, 0, unroll=False)
        pltpu.make_async_copy(bp0_hbm.at[pl.ds(0, 2 * rows_per_w)],
                              ibuf.at[pl.ds(0, 2 * rows_per_w)],
                              sem_a).wait()

        def gat(b, carry):
            s0 = pl.ds(b * slot + 3 * CHUNK, CHUNK)
            s1 = pl.ds(b * slot + 4 * CHUNK, CHUNK)
            pltpu.make_async_copy(nbr0_hbm.at[ibuf.at[s1]],
                                  ibuf.at[pl.ds(b * slot, CHUNK)],
                                  sem_g).start()
            pltpu.make_async_copy(nbr1_hbm.at[ibuf.at[s0]],
                                  ibuf.at[pl.ds(b * slot + CHUNK, CHUNK)],
                                  sem_g).start()
            pltpu.make_async_copy(nbr1_hbm.at[ibuf.at[s1]],
                                  ibuf.at[pl.ds(b * slot + 2 * CHUNK,
                                                CHUNK)], sem_g).start()
            return carry

        lax.fori_loop(0, n_blk, gat, 0, unroll=False)
        pltpu.make_async_copy(nbr0_hbm.at[pl.ds(0, 3 * rows_per_w)],
                              ibuf.at[pl.ds(0, 3 * rows_per_w)],
                              sem_g).wait()
        pltpu.sync_copy(ibuf, idx5_hbm.at[pl.ds(wid * words_per_w,
                                                words_per_w)])

    return idx_kernel(nbr0, nbr1, bp0, bp1)


def _sc_gather_sum(p1, p2, p3, q4, q5, idx5, row_off, n_rows, CHUNK):
    rows_per_w = n_rows // NW
    n_chunks = rows_per_w // CHUNK
    assert rows_per_w % CHUNK == 0 and n_chunks >= 4
    assert rows_per_w % 8 == 0 and CHUNK % 8 == 0 and row_off % 8 == 0
    mesh = plsc.VectorSubcoreMesh(core_axis_name="c", subcore_axis_name="s")

    @functools.partial(
        pl.kernel,
        mesh=mesh,
        out_type=jax.ShapeDtypeStruct((n_rows, F), jnp.float32),
        scratch_types=[
            pltpu.VMEM((5 * CHUNK,), jnp.int32),
            pltpu.VMEM((5 * CHUNK,), jnp.int32),
            pltpu.VMEM((CHUNK, F), jnp.float32),
            pltpu.VMEM((CHUNK, F), jnp.float32),
            pltpu.VMEM((CHUNK, F), jnp.float32),
            pltpu.VMEM((CHUNK, F), jnp.float32),
            pltpu.VMEM((CHUNK, F), jnp.float32),
            pltpu.VMEM((CHUNK, F), jnp.float32),
            pltpu.VMEM((CHUNK, F), jnp.float32),
            pltpu.VMEM((CHUNK, F), jnp.float32),
            pltpu.VMEM((CHUNK, F), jnp.float32),
            pltpu.VMEM((CHUNK, F), jnp.float32),
            pltpu.SemaphoreType.DMA,
            pltpu.SemaphoreType.DMA,
            pltpu.SemaphoreType.DMA,
            pltpu.SemaphoreType.DMA,
            pltpu.SemaphoreType.DMA,
            pltpu.SemaphoreType.DMA,
        ],
    )
    def sc_kernel(p1_hbm, p2_hbm, p3_hbm, q4_hbm, q5_hbm,
                  idx5_hbm, out_hbm,
                  ib0, ib1, r00, r01, r02, r03, r04,
                  r10, r11, r12, r13, r14,
                  si0, si1, sg0, sg1, so0, so1):
        wid = lax.axis_index("s") * 2 + lax.axis_index("c")
        w_base = row_off + wid * rows_per_w
        w_out = wid * rows_per_w
        ibs = [ib0, ib1]
        rbs = [[r00, r01, r02, r03, r04], [r10, r11, r12, r13, r14]]
        sis = [si0, si1]
        sgs = [sg0, sg1]
        sos = [so0, so1]
        tabs = [p1_hbm, p2_hbm, p3_hbm, q4_hbm, q5_hbm]
        slot = 5 * CHUNK
        w_slot = (w_base // CHUNK) * slot

        def issue_idx(ci, p):
            pltpu.make_async_copy(
                idx5_hbm.at[pl.ds(w_slot + ci * slot, slot)],
                ibs[p], sis[p]).start()

        def wait_idx(p):
            pltpu.make_async_copy(
                idx5_hbm.at[pl.ds(0, slot)], ibs[p], sis[p]).wait()

        def issue_gathers(p):
            for k in range(5):
                pltpu.make_async_copy(
                    tabs[k].at[ibs[p].at[pl.ds(k * CHUNK, CHUNK)]],
                    rbs[p][k], sgs[p]).start()

        def wait_gathers(p):
            for k in range(5):
                pltpu.make_async_copy(
                    tabs[k].at[pl.ds(0, CHUNK)], rbs[p][k], sgs[p]).wait()

        def accum_and_out(ci, p):
            r0, r1, r2, r3, r4 = rbs[p]

            def body(rr, c):
                for cc in range(F // 16):
                    sl = pl.ds(cc * 16, 16)
                    plsc.addupdate(r0.at[rr, sl],
                                   r1[rr, sl] + r2[rr, sl])
                    plsc.addupdate(r0.at[rr, sl],
                                   r3[rr, sl] + r4[rr, sl])
                return c

            lax.fori_loop(0, CHUNK, body, 0, unroll=False)
            base = w_out + ci * CHUNK
            pltpu.make_async_copy(r0, out_hbm.at[pl.ds(base, CHUNK)],
                                  sos[p]).start()

        def wait_out(p):
            pltpu.make_async_copy(
                rbs[p][0], out_hbm.at[pl.ds(0, CHUNK)], sos[p]).wait()

        def step(ci, p, first, do_wait_idx, issue_next_g, issue_next2_i):
            wait_gathers(p)
            if do_wait_idx:
                wait_idx(1 - p)
            if not first:
                wait_out(1 - p)
            if issue_next_g:
                issue_gathers(1 - p)
            if issue_next2_i:
                issue_idx(ci + 2, p)
            accum_and_out(ci, p)

        issue_idx(0, 0)
        wait_idx(0)
        issue_gathers(0)
        issue_idx(1, 1)
        step(0, 0, True, True, True, True)

        def pair_body(t, carry):
            a = 2 * t + 1
            step(a, 1, False, True, True, True)
            step(a + 1, 0, False, True, True, True)
            return carry

        lax.fori_loop(0, (n_chunks - 3) // 2, pair_body, 0, unroll=False)
        for ci in range(2 * ((n_chunks - 3) // 2) + 1, n_chunks):
            step(ci, ci & 1, False, ci + 1 < n_chunks, ci + 1 < n_chunks,
                 ci + 2 < n_chunks)
        wait_out((n_chunks - 1) & 1)

    return sc_kernel(p1, p2, p3, q4, q5, idx5)


def _final_body_aliased(prev_ref, h_ref, edge_ref, tdij_ref, tdik_ref, a_ref,
                        w1_ref, b1_ref, w2_ref, b2_ref, wbj_ref, bbj_ref,
                        wbk_ref, bbk_ref, wa_ref, ba_ref, bfea_ref, out_ref):
    del prev_ref
    _final_body(h_ref, edge_ref, tdij_ref, tdik_ref, a_ref,
                w1_ref, b1_ref, w2_ref, b2_ref, wbj_ref, bbj_ref,
                wbk_ref, bbk_ref, wa_ref, ba_ref, bfea_ref, out_ref)


def _final_body(h_ref, edge_ref, tdij_ref, tdik_ref, a_ref,
                w1_ref, b1_ref, w2_ref, b2_ref, wbj_ref, bbj_ref,
                wbk_ref, bbk_ref, wa_ref, ba_ref, bfea_ref, out_ref):
    h = _silu(h_ref[...] + bfea_ref[...])
    g = jax.nn.sigmoid(
        jnp.dot(h, w1_ref[...], preferred_element_type=jnp.float32)
        + b1_ref[...])
    g = g * _silu(
        jnp.dot(h, w2_ref[...], preferred_element_type=jnp.float32)
        + b2_ref[...])
    bj = jnp.dot(tdij_ref[...], wbj_ref[...],
                 preferred_element_type=jnp.float32) + bbj_ref[...]
    bk = jnp.dot(tdik_ref[...], wbk_ref[...],
                 preferred_element_type=jnp.float32) + bbk_ref[...]
    ang = a_ref[...] * wa_ref[...] + ba_ref[...]
    out_ref[...] = edge_ref[...] + g * bj * bk * ang


def kernel(atom_fea, edge_ij, triple_dist_ij, triple_dist_ik, triple_a_jik,
           nbr_atoms, bond_pairs_indices, n_bond_pairs_bond,
           W_angle, b_angle, W_bk, b_bk, W_bj, b_bj,
           W_fea, b_fea, W_1, b_1, W_2, b_2):
    n_atoms = atom_fea.shape[0]
    m_edges = edge_ij.shape[0]
    l_trip = bond_pairs_indices.shape[0]

    wf1 = W_fea[0:F]
    wf2 = W_fea[F:2 * F]
    wf3 = W_fea[2 * F:3 * F]
    wf4 = W_fea[3 * F:4 * F]
    wf5 = W_fea[4 * F:5 * F]

    p1, p2, p3 = pl.pallas_call(
        _atoms_body,
        out_shape=[jax.ShapeDtypeStruct((n_atoms, F), jnp.float32)] * 3,
    )(atom_fea, wf1, wf2, wf3)

    BM = 2000
    grid_a = m_edges // BM
    q4, q5 = pl.pallas_call(
        _edges_body,
        grid=(grid_a,),
        in_specs=[
            pl.BlockSpec((BM, F), lambda i: (i, 0)),
            pl.BlockSpec((F, F), lambda i: (0, 0)),
            pl.BlockSpec((F, F), lambda i: (0, 0)),
        ],
        out_specs=[
            pl.BlockSpec((BM, F), lambda i: (i, 0)),
            pl.BlockSpec((BM, F), lambda i: (i, 0)),
        ],
        out_shape=[jax.ShapeDtypeStruct((m_edges, F), jnp.float32)] * 2,
    )(edge_ij, wf4, wf5)

    bp0 = bond_pairs_indices[:, 0]
    bp1 = bond_pairs_indices[:, 1]
    idx5 = _sc_index_prep(nbr_atoms[:, 0], nbr_atoms[:, 1], bp0, bp1)

    BMC = 2560
    slice_rows = [64 * NW * CHUNK, 61 * NW * CHUNK]
    a_col = triple_a_jik.reshape(l_trip, 1)
    row = lambda v: v.reshape(1, F)
    const = pl.BlockSpec((1, F), lambda i: (0, 0))
    weights = (W_1, row(b_1), W_2, row(b_2), W_bj, row(b_bj), W_bk,
               row(b_bk), W_angle, row(b_angle), row(b_fea))
    weight_specs = [
        pl.BlockSpec((F, F), lambda i: (0, 0)),
        const,
        pl.BlockSpec((F, F), lambda i: (0, 0)),
        const,
        pl.BlockSpec((16, F), lambda i: (0, 0)),
        const,
        pl.BlockSpec((16, F), lambda i: (0, 0)),
        const,
        const,
        const,
        const,
    ]
    out = None
    row_off = 0
    for s, n_rows in enumerate(slice_rows):
        h_s = _sc_gather_sum(p1, p2, p3, q4, q5, idx5,
                             row_off=row_off, n_rows=n_rows, CHUNK=CHUNK)
        off = row_off // BMC
        grid_s = n_rows // BMC
        data_specs = [
            pl.BlockSpec((BMC, F), lambda i: (i, 0)),
            pl.BlockSpec((BMC, F), lambda i, o=off: (i + o, 0)),
            pl.BlockSpec((BMC, 16), lambda i, o=off: (i + o, 0)),
            pl.BlockSpec((BMC, 16), lambda i, o=off: (i + o, 0)),
            pl.BlockSpec((BMC, 1), lambda i, o=off: (i + o, 0)),
        ]
        out_spec = pl.BlockSpec((BMC, F), lambda i, o=off: (i + o, 0))
        out_shape = jax.ShapeDtypeStruct((l_trip, F), jnp.float32)
        if s == 0:
            out = pl.pallas_call(
                _final_body,
                grid=(grid_s,),
                in_specs=data_specs + weight_specs,
                out_specs=out_spec,
                out_shape=out_shape,
            )(h_s, edge_ij, triple_dist_ij, triple_dist_ik, a_col, *weights)
        else:
            out = pl.pallas_call(
                _final_body_aliased,
                grid=(grid_s,),
                in_specs=[pl.BlockSpec(memory_space=pl.ANY)] + data_specs
                + weight_specs,
                out_specs=out_spec,
                out_shape=out_shape,
                input_output_aliases={0: 0},
            )(out, h_s, edge_ij, triple_dist_ij, triple_dist_ik, a_col,
              *weights)
        row_off += n_rows
    return out


# speedup vs baseline: 1.0006x; 1.0006x over previous
import functools

import jax
import jax.numpy as jnp
from jax import lax
from jax.experimental import pallas as pl
from jax.experimental.pallas import tpu as pltpu
from jax.experimental.pallas import tpu_sc as plsc

F = 128
NW = 32
CHUNK = 80


def _silu(x):
    return x * jax.nn.sigmoid(x)


def _atoms_body(atom_ref, w1_ref, w2_ref, w3_ref, p1_ref, p2_ref, p3_ref):
    a = atom_ref[...]
    p1_ref[...] = jnp.dot(a, w1_ref[...], preferred_element_type=jnp.float32)
    p2_ref[...] = jnp.dot(a, w2_ref[...], preferred_element_type=jnp.float32)
    p3_ref[...] = jnp.dot(a, w3_ref[...], preferred_element_type=jnp.float32)


def _edges_body(edge_ref, w4_ref, w5_ref, q4_ref, q5_ref):
    e = edge_ref[...]
    q4_ref[...] = jnp.dot(e, w4_ref[...], preferred_element_type=jnp.float32)
    q5_ref[...] = jnp.dot(e, w5_ref[...], preferred_element_type=jnp.float32)


def _sc_index_prep(nbr0, nbr1, bp0, bp1):
    L = bp0.shape[0]
    rows_per_w = L // NW
    n_blk = rows_per_w // CHUNK
    slot = 5 * CHUNK
    words_per_w = n_blk * slot
    mesh = plsc.VectorSubcoreMesh(core_axis_name="c", subcore_axis_name="s")

    @functools.partial(
        pl.kernel,
        mesh=mesh,
        out_type=jax.ShapeDtypeStruct((NW * words_per_w,), jnp.int32),
        scratch_types=[
            pltpu.VMEM((words_per_w,), jnp.int32),
            pltpu.SemaphoreType.DMA,
            pltpu.SemaphoreType.DMA,
        ],
    )
    def idx_kernel(nbr0_hbm, nbr1_hbm, bp0_hbm, bp1_hbm, idx5_hbm,
                   ibuf, sem_a, sem_g):
        wid = lax.axis_index("s") * 2 + lax.axis_index("c")
        base = wid * rows_per_w

        def cp(b, carry):
            src = pl.ds(base + b * CHUNK, CHUNK)
            pltpu.make_async_copy(bp0_hbm.at[src],
                                  ibuf.at[pl.ds(b * slot + 3 * CHUNK,
                                                CHUNK)], sem_a).start()
            pltpu.make_async_copy(bp1_hbm.at[src],
                                  ibuf.at[pl.ds(b * slot + 4 * CHUNK,
                                                CHUNK)], sem_a).start()
            return carry

        lax.fori_loop(0, n_blk, cp, 0, unroll=False)
        pltpu.make_async_copy(bp0_hbm.at[pl.ds(0, 2 * rows_per_w)],
                              ibuf.at[pl.ds(0, 2 * rows_per_w)],
                              sem_a).wait()

        def gat(b, carry):
            s0 = pl.ds(b * slot + 3 * CHUNK, CHUNK)
            s1 = pl.ds(b * slot + 4 * CHUNK, CHUNK)
            pltpu.make_async_copy(nbr0_hbm.at[ibuf.at[s1]],
                                  ibuf.at[pl.ds(b * slot, CHUNK)],
                                  sem_g).start()
            pltpu.make_async_copy(nbr1_hbm.at[ibuf.at[s0]],
                                  ibuf.at[pl.ds(b * slot + CHUNK, CHUNK)],
                                  sem_g).start()
            pltpu.make_async_copy(nbr1_hbm.at[ibuf.at[s1]],
                                  ibuf.at[pl.ds(b * slot + 2 * CHUNK,
                                                CHUNK)], sem_g).start()
            return carry

        lax.fori_loop(0, n_blk, gat, 0, unroll=False)
        pltpu.make_async_copy(nbr0_hbm.at[pl.ds(0, 3 * rows_per_w)],
                              ibuf.at[pl.ds(0, 3 * rows_per_w)],
                              sem_g).wait()
        pltpu.sync_copy(ibuf, idx5_hbm.at[pl.ds(wid * words_per_w,
                                                words_per_w)])

    return idx_kernel(nbr0, nbr1, bp0, bp1)


def _sc_gather_sum(p1, p2, p3, q4, q5, idx5, row_off, n_rows, CHUNK):
    rows_per_w = n_rows // NW
    n_chunks = rows_per_w // CHUNK
    assert rows_per_w % CHUNK == 0 and n_chunks >= 4
    assert rows_per_w % 8 == 0 and CHUNK % 8 == 0 and row_off % 8 == 0
    mesh = plsc.VectorSubcoreMesh(core_axis_name="c", subcore_axis_name="s")

    @functools.partial(
        pl.kernel,
        mesh=mesh,
        out_type=jax.ShapeDtypeStruct((n_rows, F), jnp.float32),
        scratch_types=[
            pltpu.VMEM((5 * CHUNK,), jnp.int32),
            pltpu.VMEM((5 * CHUNK,), jnp.int32),
            pltpu.VMEM((CHUNK, F), jnp.float32),
            pltpu.VMEM((CHUNK, F), jnp.float32),
            pltpu.VMEM((CHUNK, F), jnp.float32),
            pltpu.VMEM((CHUNK, F), jnp.float32),
            pltpu.VMEM((CHUNK, F), jnp.float32),
            pltpu.VMEM((CHUNK, F), jnp.float32),
            pltpu.VMEM((CHUNK, F), jnp.float32),
            pltpu.VMEM((CHUNK, F), jnp.float32),
            pltpu.VMEM((CHUNK, F), jnp.float32),
            pltpu.VMEM((CHUNK, F), jnp.float32),
            pltpu.SemaphoreType.DMA,
            pltpu.SemaphoreType.DMA,
            pltpu.SemaphoreType.DMA,
            pltpu.SemaphoreType.DMA,
            pltpu.SemaphoreType.DMA,
            pltpu.SemaphoreType.DMA,
        ],
    )
    def sc_kernel(p1_hbm, p2_hbm, p3_hbm, q4_hbm, q5_hbm,
                  idx5_hbm, out_hbm,
                  ib0, ib1, r00, r01, r02, r03, r04,
                  r10, r11, r12, r13, r14,
                  si0, si1, sg0, sg1, so0, so1):
        wid = lax.axis_index("s") * 2 + lax.axis_index("c")
        w_base = row_off + wid * rows_per_w
        w_out = wid * rows_per_w
        ibs = [ib0, ib1]
        rbs = [[r00, r01, r02, r03, r04], [r10, r11, r12, r13, r14]]
        sis = [si0, si1]
        sgs = [sg0, sg1]
        sos = [so0, so1]
        tabs = [p1_hbm, p2_hbm, p3_hbm, q4_hbm, q5_hbm]
        slot = 5 * CHUNK
        w_slot = (w_base // CHUNK) * slot

        def issue_idx(ci, p):
            pltpu.make_async_copy(
                idx5_hbm.at[pl.ds(w_slot + ci * slot, slot)],
                ibs[p], sis[p]).start()

        def wait_idx(p):
            pltpu.make_async_copy(
                idx5_hbm.at[pl.ds(0, slot)], ibs[p], sis[p]).wait()

        H = CHUNK // 2

        def issue_gathers(p):
            for k in range(5):
                pltpu.make_async_copy(
                    tabs[k].at[ibs[p].at[pl.ds(k * CHUNK, H)]],
                    rbs[p][k].at[pl.ds(0, H)], sgs[p]).start()
                pltpu.make_async_copy(
                    tabs[k].at[ibs[p].at[pl.ds(k * CHUNK + H, H)]],
                    rbs[p][k].at[pl.ds(H, H)], sgs[p]).start()

        def wait_gathers(p):
            for k in range(5):
                pltpu.make_async_copy(
                    tabs[k].at[pl.ds(0, CHUNK)], rbs[p][k], sgs[p]).wait()

        def accum_and_out(ci, p):
            r0, r1, r2, r3, r4 = rbs[p]

            def body(rr, c):
                for cc in range(F // 16):
                    sl = pl.ds(cc * 16, 16)
                    plsc.addupdate(r0.at[rr, sl],
                                   r1[rr, sl] + r2[rr, sl])
                    plsc.addupdate(r0.at[rr, sl],
                                   r3[rr, sl] + r4[rr, sl])
                return c

            lax.fori_loop(0, CHUNK, body, 0, unroll=False)
            base = w_out + ci * CHUNK
            pltpu.make_async_copy(r0, out_hbm.at[pl.ds(base, CHUNK)],
                                  sos[p]).start()

        def wait_out(p):
            pltpu.make_async_copy(
                rbs[p][0], out_hbm.at[pl.ds(0, CHUNK)], sos[p]).wait()

        def step(ci, p, first, do_wait_idx, issue_next_g, issue_next2_i):
            wait_gathers(p)
            if do_wait_idx:
                wait_idx(1 - p)
            if not first:
                wait_out(1 - p)
            if issue_next_g:
                issue_gathers(1 - p)
            if issue_next2_i:
                issue_idx(ci + 2, p)
            accum_and_out(ci, p)

        issue_idx(0, 0)
        wait_idx(0)
        issue_gathers(0)
        issue_idx(1, 1)
        step(0, 0, True, True, True, True)

        def pair_body(t, carry):
            a = 2 * t + 1
            step(a, 1, False, True, True, True)
            step(a + 1, 0, False, True, True, True)
            return carry

        lax.fori_loop(0, (n_chunks - 3) // 2, pair_body, 0, unroll=False)
        for ci in range(2 * ((n_chunks - 3) // 2) + 1, n_chunks):
            step(ci, ci & 1, False, ci + 1 < n_chunks, ci + 1 < n_chunks,
                 ci + 2 < n_chunks)
        wait_out((n_chunks - 1) & 1)

    return sc_kernel(p1, p2, p3, q4, q5, idx5)


def _final_body_aliased(prev_ref, h_ref, edge_ref, tdij_ref, tdik_ref, a_ref,
                        w1_ref, b1_ref, w2_ref, b2_ref, wbj_ref, bbj_ref,
                        wbk_ref, bbk_ref, wa_ref, ba_ref, bfea_ref, out_ref):
    del prev_ref
    _final_body(h_ref, edge_ref, tdij_ref, tdik_ref, a_ref,
                w1_ref, b1_ref, w2_ref, b2_ref, wbj_ref, bbj_ref,
                wbk_ref, bbk_ref, wa_ref, ba_ref, bfea_ref, out_ref)


def _final_body(h_ref, edge_ref, tdij_ref, tdik_ref, a_ref,
                w1_ref, b1_ref, w2_ref, b2_ref, wbj_ref, bbj_ref,
                wbk_ref, bbk_ref, wa_ref, ba_ref, bfea_ref, out_ref):
    h = _silu(h_ref[...] + bfea_ref[...])
    g = jax.nn.sigmoid(
        jnp.dot(h, w1_ref[...], preferred_element_type=jnp.float32)
        + b1_ref[...])
    g = g * _silu(
        jnp.dot(h, w2_ref[...], preferred_element_type=jnp.float32)
        + b2_ref[...])
    bj = jnp.dot(tdij_ref[...], wbj_ref[...],
                 preferred_element_type=jnp.float32) + bbj_ref[...]
    bk = jnp.dot(tdik_ref[...], wbk_ref[...],
                 preferred_element_type=jnp.float32) + bbk_ref[...]
    ang = a_ref[...] * wa_ref[...] + ba_ref[...]
    out_ref[...] = edge_ref[...] + g * bj * bk * ang


def kernel(atom_fea, edge_ij, triple_dist_ij, triple_dist_ik, triple_a_jik,
           nbr_atoms, bond_pairs_indices, n_bond_pairs_bond,
           W_angle, b_angle, W_bk, b_bk, W_bj, b_bj,
           W_fea, b_fea, W_1, b_1, W_2, b_2):
    n_atoms = atom_fea.shape[0]
    m_edges = edge_ij.shape[0]
    l_trip = bond_pairs_indices.shape[0]

    wf1 = W_fea[0:F]
    wf2 = W_fea[F:2 * F]
    wf3 = W_fea[2 * F:3 * F]
    wf4 = W_fea[3 * F:4 * F]
    wf5 = W_fea[4 * F:5 * F]

    p1, p2, p3 = pl.pallas_call(
        _atoms_body,
        out_shape=[jax.ShapeDtypeStruct((n_atoms, F), jnp.float32)] * 3,
    )(atom_fea, wf1, wf2, wf3)

    BM = 2000
    grid_a = m_edges // BM
    q4, q5 = pl.pallas_call(
        _edges_body,
        grid=(grid_a,),
        in_specs=[
            pl.BlockSpec((BM, F), lambda i: (i, 0)),
            pl.BlockSpec((F, F), lambda i: (0, 0)),
            pl.BlockSpec((F, F), lambda i: (0, 0)),
        ],
        out_specs=[
            pl.BlockSpec((BM, F), lambda i: (i, 0)),
            pl.BlockSpec((BM, F), lambda i: (i, 0)),
        ],
        out_shape=[jax.ShapeDtypeStruct((m_edges, F), jnp.float32)] * 2,
    )(edge_ij, wf4, wf5)

    bp0 = bond_pairs_indices[:, 0]
    bp1 = bond_pairs_indices[:, 1]
    idx5 = _sc_index_prep(nbr_atoms[:, 0], nbr_atoms[:, 1], bp0, bp1)

    BMC = 2560
    slice_rows = [64 * NW * CHUNK, 61 * NW * CHUNK]
    a_col = triple_a_jik.reshape(l_trip, 1)
    row = lambda v: v.reshape(1, F)
    const = pl.BlockSpec((1, F), lambda i: (0, 0))
    weights = (W_1, row(b_1), W_2, row(b_2), W_bj, row(b_bj), W_bk,
               row(b_bk), W_angle, row(b_angle), row(b_fea))
    weight_specs = [
        pl.BlockSpec((F, F), lambda i: (0, 0)),
        const,
        pl.BlockSpec((F, F), lambda i: (0, 0)),
        const,
        pl.BlockSpec((16, F), lambda i: (0, 0)),
        const,
        pl.BlockSpec((16, F), lambda i: (0, 0)),
        const,
        const,
        const,
        const,
    ]
    out = None
    row_off = 0
    for s, n_rows in enumerate(slice_rows):
        h_s = _sc_gather_sum(p1, p2, p3, q4, q5, idx5,
                             row_off=row_off, n_rows=n_rows, CHUNK=CHUNK)
        off = row_off // BMC
        grid_s = n_rows // BMC
        data_specs = [
            pl.BlockSpec((BMC, F), lambda i: (i, 0)),
            pl.BlockSpec((BMC, F), lambda i, o=off: (i + o, 0)),
            pl.BlockSpec((BMC, 16), lambda i, o=off: (i + o, 0)),
            pl.BlockSpec((BMC, 16), lambda i, o=off: (i + o, 0)),
            pl.BlockSpec((BMC, 1), lambda i, o=off: (i + o, 0)),
        ]
        out_spec = pl.BlockSpec((BMC, F), lambda i, o=off: (i + o, 0))
        out_shape = jax.ShapeDtypeStruct((l_trip, F), jnp.float32)
        if s == 0:
            out = pl.pallas_call(
                _final_body,
                grid=(grid_s,),
                in_specs=data_specs + weight_specs,
                out_specs=out_spec,
                out_shape=out_shape,
            )(h_s, edge_ij, triple_dist_ij, triple_dist_ik, a_col, *weights)
        else:
            out = pl.pallas_call(
                _final_body_aliased,
                grid=(grid_s,),
                in_specs=[pl.BlockSpec(memory_space=pl.ANY)] + data_specs
                + weight_specs,
                out_specs=out_spec,
                out_shape=out_shape,
                input_output_aliases={0: 0},
            )(out, h_s, edge_ij, triple_dist_ij, triple_dist_ik, a_col,
              *weights)
        row_off += n_rows
    return out


# speedup vs baseline: 1.0026x; 1.0021x over previous
import functools

import jax
import jax.numpy as jnp
from jax import lax
from jax.experimental import pallas as pl
from jax.experimental.pallas import tpu as pltpu
from jax.experimental.pallas import tpu_sc as plsc

F = 128
NW = 32
CHUNK = 80


def _silu(x):
    return x * jax.nn.sigmoid(x)


def _atoms_body(atom_ref, w1_ref, w2_ref, w3_ref, p1_ref, p2_ref, p3_ref):
    a = atom_ref[...]
    p1_ref[...] = jnp.dot(a, w1_ref[...], preferred_element_type=jnp.float32)
    p2_ref[...] = jnp.dot(a, w2_ref[...], preferred_element_type=jnp.float32)
    p3_ref[...] = jnp.dot(a, w3_ref[...], preferred_element_type=jnp.float32)


def _edges_body(edge_ref, w4_ref, w5_ref, q4_ref, q5_ref):
    e = edge_ref[...]
    q4_ref[...] = jnp.dot(e, w4_ref[...], preferred_element_type=jnp.float32)
    q5_ref[...] = jnp.dot(e, w5_ref[...], preferred_element_type=jnp.float32)


def _sc_index_prep(nbr0, nbr1, bp0, bp1):
    L = bp0.shape[0]
    rows_per_w = L // NW
    n_blk = rows_per_w // CHUNK
    slot = 5 * CHUNK
    words_per_w = n_blk * slot
    mesh = plsc.VectorSubcoreMesh(core_axis_name="c", subcore_axis_name="s")

    @functools.partial(
        pl.kernel,
        mesh=mesh,
        out_type=jax.ShapeDtypeStruct((NW * words_per_w,), jnp.int32),
        scratch_types=[
            pltpu.VMEM((words_per_w,), jnp.int32),
            pltpu.SemaphoreType.DMA,
            pltpu.SemaphoreType.DMA,
        ],
    )
    def idx_kernel(nbr0_hbm, nbr1_hbm, bp0_hbm, bp1_hbm, idx5_hbm,
                   ibuf, sem_a, sem_g):
        wid = lax.axis_index("s") * 2 + lax.axis_index("c")
        base = wid * rows_per_w

        def cp(b, carry):
            src = pl.ds(base + b * CHUNK, CHUNK)
            pltpu.make_async_copy(bp0_hbm.at[src],
                                  ibuf.at[pl.ds(b * slot + 3 * CHUNK,
                                                CHUNK)], sem_a).start()
            pltpu.make_async_copy(bp1_hbm.at[src],
                                  ibuf.at[pl.ds(b * slot + 4 * CHUNK,
                                                CHUNK)], sem_a).start()
            return carry

        lax.fori_loop(0, n_blk, cp, 0, unroll=False)
        pltpu.make_async_copy(bp0_hbm.at[pl.ds(0, 2 * rows_per_w)],
                              ibuf.at[pl.ds(0, 2 * rows_per_w)],
                              sem_a).wait()

        def gat(b, carry):
            s0 = pl.ds(b * slot + 3 * CHUNK, CHUNK)
            s1 = pl.ds(b * slot + 4 * CHUNK, CHUNK)
            pltpu.make_async_copy(nbr0_hbm.at[ibuf.at[s1]],
                                  ibuf.at[pl.ds(b * slot, CHUNK)],
                                  sem_g).start()
            pltpu.make_async_copy(nbr1_hbm.at[ibuf.at[s0]],
                                  ibuf.at[pl.ds(b * slot + CHUNK, CHUNK)],
                                  sem_g).start()
            pltpu.make_async_copy(nbr1_hbm.at[ibuf.at[s1]],
                                  ibuf.at[pl.ds(b * slot + 2 * CHUNK,
                                                CHUNK)], sem_g).start()
            return carry

        lax.fori_loop(0, n_blk, gat, 0, unroll=False)
        pltpu.make_async_copy(nbr0_hbm.at[pl.ds(0, 3 * rows_per_w)],
                              ibuf.at[pl.ds(0, 3 * rows_per_w)],
                              sem_g).wait()
        pltpu.sync_copy(ibuf, idx5_hbm.at[pl.ds(wid * words_per_w,
                                                words_per_w)])

    return idx_kernel(nbr0, nbr1, bp0, bp1)


def _sc_gather_sum(p1, p2, p3, q4, q5, idx5, row_off, n_rows, CHUNK):
    rows_per_w = n_rows // NW
    n_chunks = rows_per_w // CHUNK
    assert rows_per_w % CHUNK == 0 and n_chunks >= 5
    assert (n_chunks - 4) % 3 == 0
    assert rows_per_w % 8 == 0 and CHUNK % 8 == 0 and row_off % 8 == 0
    mesh = plsc.VectorSubcoreMesh(core_axis_name="c", subcore_axis_name="s")

    @functools.partial(
        pl.kernel,
        mesh=mesh,
        out_type=jax.ShapeDtypeStruct((n_rows, F), jnp.float32),
        scratch_types=(
            [pltpu.VMEM((5 * CHUNK,), jnp.int32)] * 3
            + [pltpu.VMEM((CHUNK, F), jnp.float32)] * 3
            + [pltpu.SemaphoreType.DMA] * 12
        ),
    )
    def sc_kernel(p1_hbm, p2_hbm, p3_hbm, q4_hbm, q5_hbm,
                  idx5_hbm, out_hbm,
                  ib0, ib1, ib2, ab0, ab1, ab2,
                  si0, si1, si2, s00, s01, s02,
                  sa0, sa1, sa2, so0, so1, so2):
        wid = lax.axis_index("s") * 2 + lax.axis_index("c")
        w_base = row_off + wid * rows_per_w
        w_out = wid * rows_per_w
        ibs = [ib0, ib1, ib2]
        abs_ = [ab0, ab1, ab2]
        sis = [si0, si1, si2]
        s0s = [s00, s01, s02]
        sas = [sa0, sa1, sa2]
        sos = [so0, so1, so2]
        tabs = [p1_hbm, p2_hbm, p3_hbm, q4_hbm, q5_hbm]
        slot = 5 * CHUNK
        w_slot = (w_base // CHUNK) * slot

        def issue_idx(ci, r):
            pltpu.make_async_copy(
                idx5_hbm.at[pl.ds(w_slot + ci * slot, slot)],
                ibs[r], sis[r]).start()

        def wait_idx(r):
            pltpu.make_async_copy(
                idx5_hbm.at[pl.ds(0, slot)], ibs[r], sis[r]).wait()

        def issue_g0(r):
            pltpu.make_async_copy(
                tabs[0].at[ibs[r].at[pl.ds(0, CHUNK)]], abs_[r],
                s0s[r]).start()

        def wait_g0(r):
            pltpu.make_async_copy(
                tabs[0].at[pl.ds(0, CHUNK)], abs_[r], s0s[r]).wait()

        def issue_gadds(r):
            for k in range(1, 5):
                pltpu.async_copy(
                    tabs[k].at[ibs[r].at[pl.ds(k * CHUNK, CHUNK)]],
                    abs_[r], sas[r], add=True)

        def wait_gadds(r):
            for k in range(1, 5):
                pltpu.make_async_copy(
                    tabs[0].at[pl.ds(0, CHUNK)], abs_[r], sas[r]).wait()

        def issue_out(ci, r):
            pltpu.make_async_copy(
                abs_[r], out_hbm.at[pl.ds(w_out + ci * CHUNK, CHUNK)],
                sos[r]).start()

        def wait_out(r):
            pltpu.make_async_copy(
                abs_[r], out_hbm.at[pl.ds(0, CHUNK)], sos[r]).wait()

        def step(ci, r, w_i1=True, w_o2=True, g_g1=True, w_a1=True,
                 o_o1=True, g_i2=True):
            r1 = (r + 1) % 3
            r2 = (r + 2) % 3
            wait_g0(r)
            issue_gadds(r)
            if w_i1:
                wait_idx(r1)
            if w_o2:
                wait_out(r1)
            if g_g1:
                issue_g0(r1)
            if w_a1:
                wait_gadds(r2)
            if o_o1:
                issue_out(ci - 1, r2)
            if g_i2:
                issue_idx(ci + 2, r2)

        n = n_chunks
        issue_idx(0, 0)
        issue_idx(1, 1)
        wait_idx(0)
        issue_g0(0)
        step(0, 0, w_o2=False, w_a1=False, o_o1=False)
        step(1, 1, w_o2=False)

        def triple_body(t, carry):
            ci = 3 * t + 2
            step(ci, 2)
            step(ci + 1, 0)
            step(ci + 2, 1)
            return carry

        lax.fori_loop(0, (n - 4) // 3, triple_body, 0, unroll=False)
        step(n - 2, 2, g_i2=False)
        step(n - 1, 0, w_i1=False, g_g1=False, g_i2=False)
        wait_gadds(0)
        issue_out(n - 1, 0)
        wait_out(2)
        wait_out(0)

    return sc_kernel(p1, p2, p3, q4, q5, idx5)


def _final_body_aliased(prev_ref, h_ref, edge_ref, tdij_ref, tdik_ref, a_ref,
                        w1_ref, b1_ref, w2_ref, b2_ref, wbj_ref, bbj_ref,
                        wbk_ref, bbk_ref, wa_ref, ba_ref, bfea_ref, out_ref):
    del prev_ref
    _final_body(h_ref, edge_ref, tdij_ref, tdik_ref, a_ref,
                w1_ref, b1_ref, w2_ref, b2_ref, wbj_ref, bbj_ref,
                wbk_ref, bbk_ref, wa_ref, ba_ref, bfea_ref, out_ref)


def _final_body(h_ref, edge_ref, tdij_ref, tdik_ref, a_ref,
                w1_ref, b1_ref, w2_ref, b2_ref, wbj_ref, bbj_ref,
                wbk_ref, bbk_ref, wa_ref, ba_ref, bfea_ref, out_ref):
    h = _silu(h_ref[...] + bfea_ref[...])
    g = jax.nn.sigmoid(
        jnp.dot(h, w1_ref[...], preferred_element_type=jnp.float32)
        + b1_ref[...])
    g = g * _silu(
        jnp.dot(h, w2_ref[...], preferred_element_type=jnp.float32)
        + b2_ref[...])
    bj = jnp.dot(tdij_ref[...], wbj_ref[...],
                 preferred_element_type=jnp.float32) + bbj_ref[...]
    bk = jnp.dot(tdik_ref[...], wbk_ref[...],
                 preferred_element_type=jnp.float32) + bbk_ref[...]
    ang = a_ref[...] * wa_ref[...] + ba_ref[...]
    out_ref[...] = edge_ref[...] + g * bj * bk * ang


def kernel(atom_fea, edge_ij, triple_dist_ij, triple_dist_ik, triple_a_jik,
           nbr_atoms, bond_pairs_indices, n_bond_pairs_bond,
           W_angle, b_angle, W_bk, b_bk, W_bj, b_bj,
           W_fea, b_fea, W_1, b_1, W_2, b_2):
    n_atoms = atom_fea.shape[0]
    m_edges = edge_ij.shape[0]
    l_trip = bond_pairs_indices.shape[0]

    wf1 = W_fea[0:F]
    wf2 = W_fea[F:2 * F]
    wf3 = W_fea[2 * F:3 * F]
    wf4 = W_fea[3 * F:4 * F]
    wf5 = W_fea[4 * F:5 * F]

    p1, p2, p3 = pl.pallas_call(
        _atoms_body,
        out_shape=[jax.ShapeDtypeStruct((n_atoms, F), jnp.float32)] * 3,
    )(atom_fea, wf1, wf2, wf3)

    BM = 2000
    grid_a = m_edges // BM
    q4, q5 = pl.pallas_call(
        _edges_body,
        grid=(grid_a,),
        in_specs=[
            pl.BlockSpec((BM, F), lambda i: (i, 0)),
            pl.BlockSpec((F, F), lambda i: (0, 0)),
            pl.BlockSpec((F, F), lambda i: (0, 0)),
        ],
        out_specs=[
            pl.BlockSpec((BM, F), lambda i: (i, 0)),
            pl.BlockSpec((BM, F), lambda i: (i, 0)),
        ],
        out_shape=[jax.ShapeDtypeStruct((m_edges, F), jnp.float32)] * 2,
    )(edge_ij, wf4, wf5)

    bp0 = bond_pairs_indices[:, 0]
    bp1 = bond_pairs_indices[:, 1]
    idx5 = _sc_index_prep(nbr_atoms[:, 0], nbr_atoms[:, 1], bp0, bp1)

    BMC = 2560
    slice_rows = [64 * NW * CHUNK, 61 * NW * CHUNK]
    a_col = triple_a_jik.reshape(l_trip, 1)
    row = lambda v: v.reshape(1, F)
    const = pl.BlockSpec((1, F), lambda i: (0, 0))
    weights = (W_1, row(b_1), W_2, row(b_2), W_bj, row(b_bj), W_bk,
               row(b_bk), W_angle, row(b_angle), row(b_fea))
    weight_specs = [
        pl.BlockSpec((F, F), lambda i: (0, 0)),
        const,
        pl.BlockSpec((F, F), lambda i: (0, 0)),
        const,
        pl.BlockSpec((16, F), lambda i: (0, 0)),
        const,
        pl.BlockSpec((16, F), lambda i: (0, 0)),
        const,
        const,
        const,
        const,
    ]
    out = None
    row_off = 0
    for s, n_rows in enumerate(slice_rows):
        h_s = _sc_gather_sum(p1, p2, p3, q4, q5, idx5,
                             row_off=row_off, n_rows=n_rows, CHUNK=CHUNK)
        off = row_off // BMC
        grid_s = n_rows // BMC
        data_specs = [
            pl.BlockSpec((BMC, F), lambda i: (i, 0)),
            pl.BlockSpec((BMC, F), lambda i, o=off: (i + o, 0)),
            pl.BlockSpec((BMC, 16), lambda i, o=off: (i + o, 0)),
            pl.BlockSpec((BMC, 16), lambda i, o=off: (i + o, 0)),
            pl.BlockSpec((BMC, 1), lambda i, o=off: (i + o, 0)),
        ]
        out_spec = pl.BlockSpec((BMC, F), lambda i, o=off: (i + o, 0))
        out_shape = jax.ShapeDtypeStruct((l_trip, F), jnp.float32)
        if s == 0:
            out = pl.pallas_call(
                _final_body,
                grid=(grid_s,),
                in_specs=data_specs + weight_specs,
                out_specs=out_spec,
                out_shape=out_shape,
            )(h_s, edge_ij, triple_dist_ij, triple_dist_ik, a_col, *weights)
        else:
            out = pl.pallas_call(
                _final_body_aliased,
                grid=(grid_s,),
                in_specs=[pl.BlockSpec(memory_space=pl.ANY)] + data_specs
                + weight_specs,
                out_specs=out_spec,
                out_shape=out_shape,
                input_output_aliases={0: 0},
            )(out, h_s, edge_ij, triple_dist_ij, triple_dist_ik, a_col,
              *weights)
        row_off += n_rows
    return out
